# Initial kernel scaffold; baseline (speedup 1.0000x reference)
import jax
import jax.numpy as jnp
from jax.experimental import pallas as pl


def kernel(x_cont, x_cat, edge_index, batch, embed_charge, embed_pdgid, W_cont, b_cont, W_cat, b_cat, W_all, b_all, g_all, be_all, W_msg, b_msg, g_conv, be_conv, W_out1, b_out1, W_out2, b_out2):
    raise NotImplementedError("write your pallas kernel here")


# speedup vs baseline: 2.8584x; 2.8584x over previous
import functools

import jax
import jax.numpy as jnp
from jax import lax
from jax.experimental import pallas as pl
from jax.experimental.pallas import tpu as pltpu
from jax.experimental.pallas import tpu_sc as plsc

_PDGS = (1, 2, 11, 13, 22, 130, 211)
_NEG = -3.0e38
_BLK = 4000
_NWORK = 32
_CHUNK = 1600
_SUB = 256


def _elu(x):
    return jnp.where(x > 0, x, jnp.exp(x) - 1.0)


def _f32(x):
    return jnp.dot(x[0], x[1], preferred_element_type=jnp.float32)


def _dot(a, b):
    return jnp.dot(a, b, preferred_element_type=jnp.float32)


def _s1_body(xc_ref, cat_ref, ech_ref, epd_ref, wc_ref, bc_ref, wk_ref,
             bk_ref, wa_ref, ba_ref, z_ref, s_ref, ss_ref):
    h4 = ech_ref.shape[1]
    h2 = wc_ref.shape[1]
    ec = _elu(_dot(xc_ref[...], wc_ref[...]) + bc_ref[...])
    cat = cat_ref[...]
    chrg = jnp.clip(cat[:, 1:2] + 1, 0, 2)
    p = jnp.abs(cat[:, 0:1])
    for i, v in enumerate(_PDGS):
        p = jnp.where(p == v, i, p)
    p = jnp.clip(p, 0, 6)
    echrg = jnp.zeros((cat.shape[0], h4), jnp.float32)
    for k in range(3):
        echrg = echrg + jnp.where(chrg == k, 1.0, 0.0) * ech_ref[k:k + 1, :]
    epdg = jnp.zeros((cat.shape[0], h4), jnp.float32)
    for k in range(7):
        epdg = epdg + jnp.where(p == k, 1.0, 0.0) * epd_ref[k:k + 1, :]
    wk = wk_ref[...]
    ecat = _elu(_dot(echrg, wk[0:h4, :]) + _dot(epdg, wk[h4:2 * h4, :])
                + bk_ref[...])
    wa = wa_ref[...]
    z = _elu(_dot(ecat, wa[0:h2, :]) + _dot(ec, wa[h2:2 * h2, :]) + ba_ref[...])
    z_ref[...] = z

    @pl.when(pl.program_id(0) == 0)
    def _():
        s_ref[...] = jnp.zeros_like(s_ref)
        ss_ref[...] = jnp.zeros_like(ss_ref)

    s_ref[...] += jnp.sum(z, axis=0, keepdims=True)
    ss_ref[...] += jnp.sum(z * z, axis=0, keepdims=True)


def _s2_body(z_ref, s1_ref, t1_ref, wm_ref, bm_ref, e_ref, a_ref, b_ref):
    hid = z_ref.shape[1]
    e = z_ref[...] * s1_ref[...] + t1_ref[...]
    wm = wm_ref[...]
    wt = wm[0:hid, :]
    wb = wm[hid:2 * hid, :]
    e_ref[...] = e
    b_ref[...] = _dot(e, wb)
    a_ref[...] = _dot(e, wt - wb) + bm_ref[...]


def _s4_body(a_ref, m_ref, s_ref, ss_ref):
    m = m_ref[...]
    agg = jnp.where(m[:, 0:1] > -1.0e37, a_ref[...] + m, 0.0)

    @pl.when(pl.program_id(0) == 0)
    def _():
        s_ref[...] = jnp.zeros_like(s_ref)
        ss_ref[...] = jnp.zeros_like(ss_ref)

    s_ref[...] += jnp.sum(agg, axis=0, keepdims=True)
    ss_ref[...] += jnp.sum(agg * agg, axis=0, keepdims=True)


def _s5_body(e_ref, a_ref, m_ref, s2_ref, t2_ref, wo1_ref, bo1_ref, wo2_ref,
             bo2_ref, o_ref):
    m = m_ref[...]
    agg = jnp.where(m[:, 0:1] > -1.0e37, a_ref[...] + m, 0.0)
    e2 = e_ref[...] + agg * s2_ref[...] + t2_ref[...]
    h = _elu(_dot(e2, wo1_ref[...]) + bo1_ref[...])
    o_ref[...] = jnp.sum(h * wo2_ref[...], axis=1, keepdims=True) + bo2_ref[...]


def _segmax_sc(b_rows, src, dst, n, hid):
    nrange = (-(-n // _NWORK) + 7) // 8 * 8
    nlast = n - (_NWORK - 1) * nrange
    e = src.shape[0]
    nch = e // _CHUNK
    nsub = (_CHUNK + _SUB - 1) // _SUB
    cap = nsub * _SUB
    h2 = hid // 2
    mesh = plsc.VectorSubcoreMesh(core_axis_name="c", subcore_axis_name="s")
    cp = pltpu.CompilerParams(needs_layout_passes=False,
                              use_tc_tiling_on_sc=False)

    @functools.partial(
        pl.kernel,
        out_type=jax.ShapeDtypeStruct((n, hid), jnp.float32),
        mesh=mesh,
        compiler_params=cp,
        scratch_types=[
            pltpu.VMEM((nrange + 1, hid), jnp.float32),
            pltpu.VMEM((_CHUNK,), jnp.int32),
            pltpu.VMEM((_CHUNK,), jnp.int32),
            pltpu.VMEM((cap,), jnp.int32),
            pltpu.VMEM((cap,), jnp.int32),
            pltpu.VMEM((_SUB, hid), jnp.float32),
        ],
    )
    def k(b_hbm, src_hbm, dst_hbm, m_hbm, acc, dbuf, sbuf, msrc, mldst, rows):
        cid = lax.axis_index("c")
        sid = lax.axis_index("s")
        wid = sid * 2 + cid
        lo = pl.multiple_of(wid * nrange, 8)
        iota = lax.iota(jnp.int32, 16)
        negv = jnp.full((16,), _NEG, jnp.float32)
        sentv = jnp.full((16,), nrange, jnp.int32)

        @pl.loop(0, nrange + 1)
        def _(r):
            acc[r, pl.ds(0, 16)] = negv
            acc[r, pl.ds(16, 16)] = negv

        @pl.loop(0, cap // 16)
        def _(g):
            msrc[pl.ds(g * 16, 16)] = g * 16 + iota

        zero16 = jnp.zeros((16,), jnp.int32)

        def chunk_body(c, carry):
            base = pl.multiple_of(c * _CHUNK, 8)
            pltpu.sync_copy(dst_hbm.at[pl.ds(base, _CHUNK)], dbuf)
            pltpu.sync_copy(src_hbm.at[pl.ds(base, _CHUNK)], sbuf)

            def group(g, cnt_vec):
                d = dbuf[pl.ds(g * 16, 16)]
                ld = d - lo
                m = (ld >= 0) & (ld < nrange)
                pos = cnt_vec + plsc.cumsum(m.astype(jnp.int32)) - 1
                s_ = sbuf[pl.ds(g * 16, 16)]
                plsc.store_scatter(msrc, [pos], s_, mask=m)
                plsc.store_scatter(mldst, [pos], ld, mask=m)
                return cnt_vec + plsc.all_reduce_population_count(m)

            cnt_vec = lax.fori_loop(0, _CHUNK // 16, group, zero16)
            cnt = jnp.max(cnt_vec)
            pad = (16 - (cnt & 15)) & 15
            plsc.store_scatter(mldst, [cnt_vec + iota], sentv,
                               mask=iota < pad)
            cntp = cnt + pad

            for kk in range(nsub):
                @pl.when(kk * _SUB < cntp)
                def _():
                    pltpu.sync_copy(b_hbm.at[msrc.at[pl.ds(kk * _SUB, _SUB)]],
                                    rows)
                    ng = (jnp.minimum(_SUB, cntp - kk * _SUB) + 15) // 16

                    def mx(j, u):
                        rb = j * 16
                        ldvec = mldst[pl.ds(kk * _SUB + rb, 16)]
                        for t in range(16):
                            ldx = ldvec[t]
                            acc[ldx, pl.ds(0, h2)] = jnp.maximum(
                                acc[ldx, pl.ds(0, h2)],
                                rows[rb + t, pl.ds(0, h2)])
                            acc[ldx, pl.ds(h2, h2)] = jnp.maximum(
                                acc[ldx, pl.ds(h2, h2)],
                                rows[rb + t, pl.ds(h2, h2)])
                        return u

                    lax.fori_loop(0, ng, mx, 0)
            return carry

        lax.fori_loop(0, nch, chunk_body, 0)

        @pl.when(wid < _NWORK - 1)
        def _():
            pltpu.sync_copy(acc.at[pl.ds(0, nrange)],
                            m_hbm.at[pl.ds(lo, nrange)])

        @pl.when(wid == _NWORK - 1)
        def _():
            pltpu.sync_copy(acc.at[pl.ds(0, nlast)],
                            m_hbm.at[pl.ds(lo, nlast)])

    return k(b_rows, src, dst)


def kernel(x_cont, x_cat, edge_index, batch, embed_charge, embed_pdgid,
           W_cont, b_cont, W_cat, b_cat, W_all, b_all, g_all, be_all,
           W_msg, b_msg, g_conv, be_conv, W_out1, b_out1, W_out2, b_out2):
    n, cont = x_cont.shape
    hid = W_all.shape[0]
    h4 = hid // 4
    h2 = hid // 2
    grid = n // _BLK
    f32 = jnp.float32

    x_cat = x_cat.astype(jnp.int32)
    src = edge_index[0].astype(jnp.int32)
    dst = edge_index[1].astype(jnp.int32)

    bc2 = b_cont.reshape(1, h2)
    bk2 = b_cat.reshape(1, h2)
    ba2 = b_all.reshape(1, hid)
    bm2 = b_msg.reshape(1, hid)
    bo1 = b_out1.reshape(1, h2)
    wo2 = W_out2.reshape(1, h2)
    bo2 = b_out2.reshape(1, 1)

    full = lambda s: pl.BlockSpec(s, lambda i: (0, 0))
    row = lambda c: pl.BlockSpec((_BLK, c), lambda i: (i, 0))

    z, s1s, s1ss = pl.pallas_call(
        _s1_body,
        grid=(grid,),
        in_specs=[row(cont), row(2), full((3, h4)), full((7, h4)),
                  full((cont, h2)), full((1, h2)), full((h2, h2)),
                  full((1, h2)), full((hid, hid)), full((1, hid))],
        out_specs=[row(hid), full((1, hid)), full((1, hid))],
        out_shape=[jax.ShapeDtypeStruct((n, hid), f32),
                   jax.ShapeDtypeStruct((1, hid), f32),
                   jax.ShapeDtypeStruct((1, hid), f32)],
    )(x_cont, x_cat, embed_charge, embed_pdgid, W_cont, bc2, W_cat, bk2,
      W_all, ba2)

    mean1 = s1s / n
    var1 = s1ss / n - mean1 * mean1
    sc1 = g_all.reshape(1, hid) / jnp.sqrt(var1 + 1e-5)
    sh1 = be_all.reshape(1, hid) - mean1 * sc1

    emb, a_rows, b_rows = pl.pallas_call(
        _s2_body,
        grid=(grid,),
        in_specs=[row(hid), full((1, hid)), full((1, hid)),
                  full((2 * hid, hid)), full((1, hid))],
        out_specs=[row(hid), row(hid), row(hid)],
        out_shape=[jax.ShapeDtypeStruct((n, hid), f32),
                   jax.ShapeDtypeStruct((n, hid), f32),
                   jax.ShapeDtypeStruct((n, hid), f32)],
    )(z, sc1, sh1, W_msg, bm2)

    m_rows = _segmax_sc(b_rows, src, dst, n, hid)

    s2s, s2ss = pl.pallas_call(
        _s4_body,
        grid=(grid,),
        in_specs=[row(hid), row(hid)],
        out_specs=[full((1, hid)), full((1, hid))],
        out_shape=[jax.ShapeDtypeStruct((1, hid), f32),
                   jax.ShapeDtypeStruct((1, hid), f32)],
    )(a_rows, m_rows)

    mean2 = s2s / n
    var2 = s2ss / n - mean2 * mean2
    sc2 = g_conv.reshape(1, hid) / jnp.sqrt(var2 + 1e-5)
    sh2 = be_conv.reshape(1, hid) - mean2 * sc2

    out = pl.pallas_call(
        _s5_body,
        grid=(grid,),
        in_specs=[row(hid), row(hid), row(hid), full((1, hid)),
                  full((1, hid)), full((hid, h2)), full((1, h2)),
                  full((1, h2)), full((1, 1))],
        out_specs=row(1),
        out_shape=jax.ShapeDtypeStruct((n, 1), f32),
    )(emb, a_rows, m_rows, sc2, sh2, W_out1, bo1, wo2, bo2)

    return out.reshape(n)


# speedup vs baseline: 3.8183x; 1.3358x over previous
import functools

import jax
import jax.numpy as jnp
from jax import lax
from jax.experimental import pallas as pl
from jax.experimental.pallas import tpu as pltpu
from jax.experimental.pallas import tpu_sc as plsc

_PDGS = (1, 2, 11, 13, 22, 130, 211)
_NEG = -3.0e38
_BLK = 4000
_NWORK = 32
_CHUNK = 1600
_SUB = 256


def _elu(x):
    return jnp.where(x > 0, x, jnp.exp(x) - 1.0)


def _f32(x):
    return jnp.dot(x[0], x[1], preferred_element_type=jnp.float32)


def _dot(a, b):
    return jnp.dot(a, b, preferred_element_type=jnp.float32)


def _s1_body(xc_ref, cat_ref, ech_ref, epd_ref, wc_ref, bc_ref, wk_ref,
             bk_ref, wa_ref, ba_ref, z_ref, s_ref, ss_ref):
    h4 = ech_ref.shape[1]
    h2 = wc_ref.shape[1]
    ec = _elu(_dot(xc_ref[...], wc_ref[...]) + bc_ref[...])
    cat = cat_ref[...]
    chrg = jnp.clip(cat[:, 1:2] + 1, 0, 2)
    p = jnp.abs(cat[:, 0:1])
    for i, v in enumerate(_PDGS):
        p = jnp.where(p == v, i, p)
    p = jnp.clip(p, 0, 6)
    echrg = jnp.zeros((cat.shape[0], h4), jnp.float32)
    for k in range(3):
        echrg = echrg + jnp.where(chrg == k, 1.0, 0.0) * ech_ref[k:k + 1, :]
    epdg = jnp.zeros((cat.shape[0], h4), jnp.float32)
    for k in range(7):
        epdg = epdg + jnp.where(p == k, 1.0, 0.0) * epd_ref[k:k + 1, :]
    wk = wk_ref[...]
    ecat = _elu(_dot(echrg, wk[0:h4, :]) + _dot(epdg, wk[h4:2 * h4, :])
                + bk_ref[...])
    wa = wa_ref[...]
    z = _elu(_dot(ecat, wa[0:h2, :]) + _dot(ec, wa[h2:2 * h2, :]) + ba_ref[...])
    z_ref[...] = z

    @pl.when(pl.program_id(0) == 0)
    def _():
        s_ref[...] = jnp.zeros_like(s_ref)
        ss_ref[...] = jnp.zeros_like(ss_ref)

    s_ref[...] += jnp.sum(z, axis=0, keepdims=True)
    ss_ref[...] += jnp.sum(z * z, axis=0, keepdims=True)


def _s2_body(z_ref, s1_ref, t1_ref, wm_ref, bm_ref, e_ref, a_ref, b_ref):
    hid = z_ref.shape[1]
    e = z_ref[...] * s1_ref[...] + t1_ref[...]
    wm = wm_ref[...]
    wt = wm[0:hid, :]
    wb = wm[hid:2 * hid, :]
    e_ref[...] = e
    b_ref[...] = _dot(e, wb)
    a_ref[...] = _dot(e, wt - wb) + bm_ref[...]


def _s4_body(a_ref, m_ref, s_ref, ss_ref):
    m = m_ref[...]
    agg = jnp.where(m[:, 0:1] > -1.0e37, a_ref[...] + m, 0.0)

    @pl.when(pl.program_id(0) == 0)
    def _():
        s_ref[...] = jnp.zeros_like(s_ref)
        ss_ref[...] = jnp.zeros_like(ss_ref)

    s_ref[...] += jnp.sum(agg, axis=0, keepdims=True)
    ss_ref[...] += jnp.sum(agg * agg, axis=0, keepdims=True)


def _s5_body(e_ref, a_ref, m_ref, s2_ref, t2_ref, wo1_ref, bo1_ref, wo2_ref,
             bo2_ref, o_ref):
    m = m_ref[...]
    agg = jnp.where(m[:, 0:1] > -1.0e37, a_ref[...] + m, 0.0)
    e2 = e_ref[...] + agg * s2_ref[...] + t2_ref[...]
    h = _elu(_dot(e2, wo1_ref[...]) + bo1_ref[...])
    o_ref[...] = jnp.sum(h * wo2_ref[...], axis=1, keepdims=True) + bo2_ref[...]


def _segmax_sc(b_rows, src, dst, n, hid):
    nrange = (-(-n // _NWORK) + 7) // 8 * 8
    nlast = n - (_NWORK - 1) * nrange
    e = src.shape[0]
    nch = e // _CHUNK
    nsub = (_CHUNK + _SUB - 1) // _SUB
    cap = nsub * _SUB
    h2 = hid // 2
    mesh = plsc.VectorSubcoreMesh(core_axis_name="c", subcore_axis_name="s")
    cp = pltpu.CompilerParams(needs_layout_passes=False,
                              use_tc_tiling_on_sc=False)

    @functools.partial(
        pl.kernel,
        out_type=jax.ShapeDtypeStruct((n, hid), jnp.float32),
        mesh=mesh,
        compiler_params=cp,
        scratch_types=[
            pltpu.VMEM((nrange + 1, hid), jnp.float32),
            pltpu.VMEM((_CHUNK,), jnp.int32),
            pltpu.VMEM((_CHUNK,), jnp.int32),
            pltpu.VMEM((_CHUNK,), jnp.int32),
            pltpu.VMEM((_CHUNK,), jnp.int32),
            pltpu.VMEM((cap,), jnp.int32),
            pltpu.VMEM((cap,), jnp.int32),
            pltpu.VMEM((_SUB, hid), jnp.float32),
            pltpu.VMEM((_SUB, hid), jnp.float32),
            pltpu.SemaphoreType.DMA,
            pltpu.SemaphoreType.DMA,
            pltpu.SemaphoreType.DMA,
            pltpu.SemaphoreType.DMA,
            pltpu.SemaphoreType.DMA,
            pltpu.SemaphoreType.DMA,
        ],
    )
    def k(b_hbm, src_hbm, dst_hbm, m_hbm, acc, dbuf0, dbuf1, sbuf0, sbuf1,
          msrc, mldst, rows0, rows1, semd0, semd1, sems0, sems1, semg0,
          semg1):
        cid = lax.axis_index("c")
        sid = lax.axis_index("s")
        wid = sid * 2 + cid
        lo = pl.multiple_of(wid * nrange, 8)
        iota = lax.iota(jnp.int32, 16)
        negv = jnp.full((16,), _NEG, jnp.float32)
        sentv = jnp.full((16,), nrange, jnp.int32)

        @pl.loop(0, nrange + 1)
        def _(r):
            acc[r, pl.ds(0, 16)] = negv
            acc[r, pl.ds(16, 16)] = negv

        @pl.loop(0, cap // 16)
        def _(g):
            msrc[pl.ds(g * 16, 16)] = g * 16 + iota

        zero16 = jnp.zeros((16,), jnp.int32)

        def idx_start(c, db, sb, semd, sems):
            base = pl.multiple_of(lax.rem(c, nch) * _CHUNK, 8)
            pltpu.make_async_copy(dst_hbm.at[pl.ds(base, _CHUNK)], db,
                                  semd).start()
            pltpu.make_async_copy(src_hbm.at[pl.ds(base, _CHUNK)], sb,
                                  sems).start()

        def idx_wait(db, sb, semd, sems):
            pltpu.make_async_copy(dst_hbm.at[pl.ds(0, _CHUNK)], db,
                                  semd).wait()
            pltpu.make_async_copy(src_hbm.at[pl.ds(0, _CHUNK)], sb,
                                  sems).wait()

        def g_copy(kk, rb, sg):
            return pltpu.make_async_copy(
                b_hbm.at[msrc.at[pl.ds(kk * _SUB, _SUB)]], rb, sg)

        rbufs = (rows0, rows1)
        gsems = (semg0, semg1)

        def process(db, sb):
            def scan4(q, cnt_vec):
                cv = cnt_vec
                for u in range(4):
                    g = q * 4 + u
                    d = db[pl.ds(g * 16, 16)]
                    ld = d - lo
                    m = (ld >= 0) & (ld < nrange)
                    pos = cv + plsc.cumsum(m.astype(jnp.int32)) - 1
                    s_ = sb[pl.ds(g * 16, 16)]
                    plsc.store_scatter(msrc, [pos], s_, mask=m)
                    plsc.store_scatter(mldst, [pos], ld, mask=m)
                    cv = cv + plsc.all_reduce_population_count(m)
                return cv

            cnt_vec = lax.fori_loop(0, _CHUNK // 64, scan4, zero16)
            cnt = jnp.max(cnt_vec)
            pad = (16 - (cnt & 15)) & 15
            plsc.store_scatter(mldst, [cnt_vec + iota], sentv,
                               mask=iota < pad)
            cntp = cnt + pad

            @pl.when(0 < cntp)
            def _():
                g_copy(0, rows0, semg0).start()

            for kk in range(nsub):
                rb_, sg_ = rbufs[kk % 2], gsems[kk % 2]
                nrb_, nsg_ = rbufs[(kk + 1) % 2], gsems[(kk + 1) % 2]

                @pl.when(kk * _SUB < cntp)
                def _(kk=kk, rb_=rb_, sg_=sg_, nrb_=nrb_, nsg_=nsg_):
                    g_copy(kk, rb_, sg_).wait()
                    if kk + 1 < nsub:
                        @pl.when((kk + 1) * _SUB < cntp)
                        def _():
                            g_copy(kk + 1, nrb_, nsg_).start()
                    ng = (jnp.minimum(_SUB, cntp - kk * _SUB) + 15) // 16

                    def mx(j, u):
                        rbase = j * 16
                        ldvec = mldst[pl.ds(kk * _SUB + rbase, 16)]
                        for t in range(16):
                            ldx = ldvec[t]
                            acc[ldx, pl.ds(0, h2)] = jnp.maximum(
                                acc[ldx, pl.ds(0, h2)],
                                rb_[rbase + t, pl.ds(0, h2)])
                            acc[ldx, pl.ds(h2, h2)] = jnp.maximum(
                                acc[ldx, pl.ds(h2, h2)],
                                rb_[rbase + t, pl.ds(h2, h2)])
                        return u

                    lax.fori_loop(0, ng, mx, 0)

        idx_start(0, dbuf0, sbuf0, semd0, sems0)

        def chunk_pair(i, carry):
            c = i * 2
            idx_start(c + 1, dbuf1, sbuf1, semd1, sems1)
            idx_wait(dbuf0, sbuf0, semd0, sems0)
            process(dbuf0, sbuf0)
            idx_start(c + 2, dbuf0, sbuf0, semd0, sems0)
            idx_wait(dbuf1, sbuf1, semd1, sems1)
            process(dbuf1, sbuf1)
            return carry

        lax.fori_loop(0, nch // 2, chunk_pair, 0)
        idx_wait(dbuf0, sbuf0, semd0, sems0)

        @pl.when(wid < _NWORK - 1)
        def _():
            pltpu.sync_copy(acc.at[pl.ds(0, nrange)],
                            m_hbm.at[pl.ds(lo, nrange)])

        @pl.when(wid == _NWORK - 1)
        def _():
            pltpu.sync_copy(acc.at[pl.ds(0, nlast)],
                            m_hbm.at[pl.ds(lo, nlast)])

    return k(b_rows, src, dst)


def kernel(x_cont, x_cat, edge_index, batch, embed_charge, embed_pdgid,
           W_cont, b_cont, W_cat, b_cat, W_all, b_all, g_all, be_all,
           W_msg, b_msg, g_conv, be_conv, W_out1, b_out1, W_out2, b_out2):
    n, cont = x_cont.shape
    hid = W_all.shape[0]
    h4 = hid // 4
    h2 = hid // 2
    grid = n // _BLK
    f32 = jnp.float32

    x_cat = x_cat.astype(jnp.int32)
    src = edge_index[0].astype(jnp.int32)
    dst = edge_index[1].astype(jnp.int32)

    bc2 = b_cont.reshape(1, h2)
    bk2 = b_cat.reshape(1, h2)
    ba2 = b_all.reshape(1, hid)
    bm2 = b_msg.reshape(1, hid)
    bo1 = b_out1.reshape(1, h2)
    wo2 = W_out2.reshape(1, h2)
    bo2 = b_out2.reshape(1, 1)

    full = lambda s: pl.BlockSpec(s, lambda i: (0, 0))
    row = lambda c: pl.BlockSpec((_BLK, c), lambda i: (i, 0))

    z, s1s, s1ss = pl.pallas_call(
        _s1_body,
        grid=(grid,),
        in_specs=[row(cont), row(2), full((3, h4)), full((7, h4)),
                  full((cont, h2)), full((1, h2)), full((h2, h2)),
                  full((1, h2)), full((hid, hid)), full((1, hid))],
        out_specs=[row(hid), full((1, hid)), full((1, hid))],
        out_shape=[jax.ShapeDtypeStruct((n, hid), f32),
                   jax.ShapeDtypeStruct((1, hid), f32),
                   jax.ShapeDtypeStruct((1, hid), f32)],
    )(x_cont, x_cat, embed_charge, embed_pdgid, W_cont, bc2, W_cat, bk2,
      W_all, ba2)

    mean1 = s1s / n
    var1 = s1ss / n - mean1 * mean1
    sc1 = g_all.reshape(1, hid) / jnp.sqrt(var1 + 1e-5)
    sh1 = be_all.reshape(1, hid) - mean1 * sc1

    emb, a_rows, b_rows = pl.pallas_call(
        _s2_body,
        grid=(grid,),
        in_specs=[row(hid), full((1, hid)), full((1, hid)),
                  full((2 * hid, hid)), full((1, hid))],
        out_specs=[row(hid), row(hid), row(hid)],
        out_shape=[jax.ShapeDtypeStruct((n, hid), f32),
                   jax.ShapeDtypeStruct((n, hid), f32),
                   jax.ShapeDtypeStruct((n, hid), f32)],
    )(z, sc1, sh1, W_msg, bm2)

    m_rows = _segmax_sc(b_rows, src, dst, n, hid)

    s2s, s2ss = pl.pallas_call(
        _s4_body,
        grid=(grid,),
        in_specs=[row(hid), row(hid)],
        out_specs=[full((1, hid)), full((1, hid))],
        out_shape=[jax.ShapeDtypeStruct((1, hid), f32),
                   jax.ShapeDtypeStruct((1, hid), f32)],
    )(a_rows, m_rows)

    mean2 = s2s / n
    var2 = s2ss / n - mean2 * mean2
    sc2 = g_conv.reshape(1, hid) / jnp.sqrt(var2 + 1e-5)
    sh2 = be_conv.reshape(1, hid) - mean2 * sc2

    out = pl.pallas_call(
        _s5_body,
        grid=(grid,),
        in_specs=[row(hid), row(hid), row(hid), full((1, hid)),
                  full((1, hid)), full((hid, h2)), full((1, h2)),
                  full((1, h2)), full((1, 1))],
        out_specs=row(1),
        out_shape=jax.ShapeDtypeStruct((n, 1), f32),
    )(emb, a_rows, m_rows, sc2, sh2, W_out1, bo1, wo2, bo2)

    return out.reshape(n)


# speedup vs baseline: 4.8752x; 1.2768x over previous
import functools

import jax
import jax.numpy as jnp
from jax import lax
from jax.experimental import pallas as pl
from jax.experimental.pallas import tpu as pltpu
from jax.experimental.pallas import tpu_sc as plsc

_PDGS = (1, 2, 11, 13, 22, 130, 211)
_NEG = -3.0e38
_BLK = 4000
_NWORK = 32
_CHUNK = 1600
_SUB = 256


def _elu(x):
    return jnp.where(x > 0, x, jnp.exp(x) - 1.0)


def _f32(x):
    return jnp.dot(x[0], x[1], preferred_element_type=jnp.float32)


def _dot(a, b):
    return jnp.dot(a, b, preferred_element_type=jnp.float32)


def _s1_body(xc_ref, cat_ref, ech_ref, epd_ref, wc_ref, bc_ref, wk_ref,
             bk_ref, wa_ref, ba_ref, z_ref, s_ref, ss_ref):
    h4 = ech_ref.shape[1]
    h2 = wc_ref.shape[1]
    ec = _elu(_dot(xc_ref[...], wc_ref[...]) + bc_ref[...])
    cat = cat_ref[...]
    chrg = jnp.clip(cat[:, 1:2] + 1, 0, 2)
    p = jnp.abs(cat[:, 0:1])
    for i, v in enumerate(_PDGS):
        p = jnp.where(p == v, i, p)
    p = jnp.clip(p, 0, 6)
    echrg = jnp.zeros((cat.shape[0], h4), jnp.float32)
    for k in range(3):
        echrg = echrg + jnp.where(chrg == k, 1.0, 0.0) * ech_ref[k:k + 1, :]
    epdg = jnp.zeros((cat.shape[0], h4), jnp.float32)
    for k in range(7):
        epdg = epdg + jnp.where(p == k, 1.0, 0.0) * epd_ref[k:k + 1, :]
    wk = wk_ref[...]
    ecat = _elu(_dot(echrg, wk[0:h4, :]) + _dot(epdg, wk[h4:2 * h4, :])
                + bk_ref[...])
    wa = wa_ref[...]
    z = _elu(_dot(ecat, wa[0:h2, :]) + _dot(ec, wa[h2:2 * h2, :]) + ba_ref[...])
    z_ref[...] = z

    @pl.when(pl.program_id(0) == 0)
    def _():
        s_ref[...] = jnp.zeros_like(s_ref)
        ss_ref[...] = jnp.zeros_like(ss_ref)

    s_ref[...] += jnp.sum(z, axis=0, keepdims=True)
    ss_ref[...] += jnp.sum(z * z, axis=0, keepdims=True)


def _s2_body(z_ref, s1_ref, t1_ref, wm_ref, bm_ref, e_ref, a_ref, b_ref):
    hid = z_ref.shape[1]
    e = z_ref[...] * s1_ref[...] + t1_ref[...]
    wm = wm_ref[...]
    wt = wm[0:hid, :]
    wb = wm[hid:2 * hid, :]
    e_ref[...] = e
    b_ref[...] = _dot(e, wb)
    a_ref[...] = _dot(e, wt - wb) + bm_ref[...]


def _s4_body(a_ref, m_ref, s_ref, ss_ref):
    m = m_ref[...]
    agg = jnp.where(m[:, 0:1] > -1.0e37, a_ref[...] + m, 0.0)

    @pl.when(pl.program_id(0) == 0)
    def _():
        s_ref[...] = jnp.zeros_like(s_ref)
        ss_ref[...] = jnp.zeros_like(ss_ref)

    s_ref[...] += jnp.sum(agg, axis=0, keepdims=True)
    ss_ref[...] += jnp.sum(agg * agg, axis=0, keepdims=True)


def _s5_body(e_ref, a_ref, m_ref, s2_ref, t2_ref, wo1_ref, bo1_ref, wo2_ref,
             bo2_ref, o_ref):
    m = m_ref[...]
    agg = jnp.where(m[:, 0:1] > -1.0e37, a_ref[...] + m, 0.0)
    e2 = e_ref[...] + agg * s2_ref[...] + t2_ref[...]
    h = _elu(_dot(e2, wo1_ref[...]) + bo1_ref[...])
    o_ref[...] = jnp.sum(h * wo2_ref[...], axis=1, keepdims=True) + bo2_ref[...]


def _segmax_sc(b_rows, src, dst, n, hid):
    nrange = (-(-n // _NWORK) + 7) // 8 * 8
    nlast = n - (_NWORK - 1) * nrange
    e = src.shape[0]
    nch = e // _CHUNK
    sbits = max(nrange.bit_length(), 1)
    ldmask = (1 << sbits) - 1
    scap = (_SUB + _CHUNK + 16 + 63) // 64 * 64
    h2 = hid // 2
    mesh = plsc.VectorSubcoreMesh(core_axis_name="c", subcore_axis_name="s")
    cp = pltpu.CompilerParams(needs_layout_passes=False,
                              use_tc_tiling_on_sc=False)

    @functools.partial(
        pl.kernel,
        out_type=jax.ShapeDtypeStruct((n, hid), jnp.float32),
        mesh=mesh,
        compiler_params=cp,
        scratch_types=[
            pltpu.VMEM((nrange + 1, hid), jnp.float32),
            pltpu.VMEM((_CHUNK,), jnp.int32),
            pltpu.VMEM((_CHUNK,), jnp.int32),
            pltpu.VMEM((_CHUNK,), jnp.int32),
            pltpu.VMEM((_CHUNK,), jnp.int32),
            pltpu.VMEM((scap,), jnp.int32),
            pltpu.VMEM((scap,), jnp.int32),
            pltpu.VMEM((_SUB, hid), jnp.float32),
            pltpu.VMEM((_SUB, hid), jnp.float32),
            pltpu.SemaphoreType.DMA,
            pltpu.SemaphoreType.DMA,
            pltpu.SemaphoreType.DMA,
            pltpu.SemaphoreType.DMA,
            pltpu.SemaphoreType.DMA,
            pltpu.SemaphoreType.DMA,
        ],
    )
    def k(b_hbm, src_hbm, dst_hbm, m_hbm, acc, dbuf0, dbuf1, sbuf0, sbuf1,
          mpack, msrc, rows0, rows1, semd0, semd1, sems0, sems1, semg0,
          semg1):
        cid = lax.axis_index("c")
        sid = lax.axis_index("s")
        wid = sid * 2 + cid
        lo = pl.multiple_of(wid * nrange, 8)
        iota = lax.iota(jnp.int32, 16)
        negv = jnp.full((16,), _NEG, jnp.float32)

        @pl.loop(0, nrange + 1)
        def _(r):
            acc[r, pl.ds(0, 16)] = negv
            acc[r, pl.ds(16, 16)] = negv

        @pl.loop(0, scap // 16)
        def _(g):
            msrc[pl.ds(g * 16, 16)] = g * 16 + iota

        zero16 = jnp.zeros((16,), jnp.int32)

        def idx_start(c, db, sb, semd, sems):
            base = pl.multiple_of(lax.rem(c, nch) * _CHUNK, 8)
            pltpu.make_async_copy(dst_hbm.at[pl.ds(base, _CHUNK)], db,
                                  semd).start()
            pltpu.make_async_copy(src_hbm.at[pl.ds(base, _CHUNK)], sb,
                                  sems).start()

        def idx_wait(db, sb, semd, sems):
            pltpu.make_async_copy(dst_hbm.at[pl.ds(0, _CHUNK)], db,
                                  semd).wait()
            pltpu.make_async_copy(src_hbm.at[pl.ds(0, _CHUNK)], sb,
                                  sems).wait()

        def g_copy(t2, rb, sg):
            t2a = pl.multiple_of(t2, 8)
            return pltpu.make_async_copy(
                b_hbm.at[msrc.at[pl.ds(t2a, _SUB)]], rb, sg)

        def maxbatch(t2, rb, ngroups):
            def mx(j, u):
                rbase = j * 16
                base = pl.multiple_of(t2 + rbase, 8)
                ldvec = mpack[pl.ds(base, 16)] & ldmask
                for t in range(16):
                    ldx = ldvec[t]
                    acc[ldx, pl.ds(0, h2)] = jnp.maximum(
                        acc[ldx, pl.ds(0, h2)],
                        rb[rbase + t, pl.ds(0, h2)])
                    acc[ldx, pl.ds(h2, h2)] = jnp.maximum(
                        acc[ldx, pl.ds(h2, h2)],
                        rb[rbase + t, pl.ds(h2, h2)])
                return u

            lax.fori_loop(0, ngroups, mx, 0)

        def process(db, sb, top):
            topv = jnp.zeros((16,), jnp.int32) + top

            def scan4(q, cv):
                for u in range(4):
                    g = q * 4 + u
                    d = db[pl.ds(g * 16, 16)]
                    ld = d - lo
                    m = (ld >= 0) & (ld < nrange)
                    pos = cv + plsc.cumsum(m.astype(jnp.int32)) - 1
                    s_ = sb[pl.ds(g * 16, 16)]
                    plsc.store_scatter(mpack, [pos], (s_ << sbits) | ld,
                                       mask=m)
                    cv = cv + plsc.all_reduce_population_count(m)
                return cv

            cv = lax.fori_loop(0, _CHUNK // 64, scan4, topv)
            tc_ = jnp.max(cv)
            pad = (16 - (tc_ & 15)) & 15
            sent = (wid << sbits) | nrange
            plsc.store_scatter(mpack, [tc_ + iota], zero16 + sent,
                               mask=iota < pad)
            topp = tc_ + pad

            def unp(g, u):
                base = pl.multiple_of(top + g * 16, 8)
                pv = mpack[pl.ds(base, 16)]
                msrc[pl.ds(base, 16)] = lax.shift_right_logical(pv, sbits)
                return u

            lax.fori_loop(0, (topp - top) // 16, unp, 0)

            def d2(t):
                ca = g_copy(t - _SUB, rows0, semg0)
                cb = g_copy(t - 2 * _SUB, rows1, semg1)
                ca.start()
                cb.start()
                ca.wait()
                maxbatch(t - _SUB, rows0, _SUB // 16)
                cb.wait()
                maxbatch(t - 2 * _SUB, rows1, _SUB // 16)
                return t - 2 * _SUB

            t2_ = lax.while_loop(lambda t: t >= 2 * _SUB, d2, topp)

            def d1(t):
                c1 = g_copy(t - _SUB, rows0, semg0)
                c1.start()
                c1.wait()
                maxbatch(t - _SUB, rows0, _SUB // 16)
                return t - _SUB

            return lax.while_loop(lambda t: t >= _SUB, d1, t2_)

        idx_start(0, dbuf0, sbuf0, semd0, sems0)

        def chunk_pair(i, top):
            c = i * 2
            idx_start(c + 1, dbuf1, sbuf1, semd1, sems1)
            idx_wait(dbuf0, sbuf0, semd0, sems0)
            top = process(dbuf0, sbuf0, top)
            idx_start(c + 2, dbuf0, sbuf0, semd0, sems0)
            idx_wait(dbuf1, sbuf1, semd1, sems1)
            top = process(dbuf1, sbuf1, top)
            return top

        top = lax.fori_loop(0, nch // 2, chunk_pair, jnp.int32(0))
        idx_wait(dbuf0, sbuf0, semd0, sems0)

        @pl.when(top > 0)
        def _():
            c1 = pltpu.make_async_copy(
                b_hbm.at[msrc.at[pl.ds(0, _SUB)]], rows0, semg0)
            c1.start()
            c1.wait()
            maxbatch(0, rows0, top // 16)

        @pl.when(wid < _NWORK - 1)
        def _():
            pltpu.sync_copy(acc.at[pl.ds(0, nrange)],
                            m_hbm.at[pl.ds(lo, nrange)])

        @pl.when(wid == _NWORK - 1)
        def _():
            pltpu.sync_copy(acc.at[pl.ds(0, nlast)],
                            m_hbm.at[pl.ds(lo, nlast)])

    return k(b_rows, src, dst)


def kernel(x_cont, x_cat, edge_index, batch, embed_charge, embed_pdgid,
           W_cont, b_cont, W_cat, b_cat, W_all, b_all, g_all, be_all,
           W_msg, b_msg, g_conv, be_conv, W_out1, b_out1, W_out2, b_out2):
    n, cont = x_cont.shape
    hid = W_all.shape[0]
    h4 = hid // 4
    h2 = hid // 2
    grid = n // _BLK
    f32 = jnp.float32

    x_cat = x_cat.astype(jnp.int32)
    src = edge_index[0].astype(jnp.int32)
    dst = edge_index[1].astype(jnp.int32)

    bc2 = b_cont.reshape(1, h2)
    bk2 = b_cat.reshape(1, h2)
    ba2 = b_all.reshape(1, hid)
    bm2 = b_msg.reshape(1, hid)
    bo1 = b_out1.reshape(1, h2)
    wo2 = W_out2.reshape(1, h2)
    bo2 = b_out2.reshape(1, 1)

    full = lambda s: pl.BlockSpec(s, lambda i: (0, 0))
    row = lambda c: pl.BlockSpec((_BLK, c), lambda i: (i, 0))

    z, s1s, s1ss = pl.pallas_call(
        _s1_body,
        grid=(grid,),
        in_specs=[row(cont), row(2), full((3, h4)), full((7, h4)),
                  full((cont, h2)), full((1, h2)), full((h2, h2)),
                  full((1, h2)), full((hid, hid)), full((1, hid))],
        out_specs=[row(hid), full((1, hid)), full((1, hid))],
        out_shape=[jax.ShapeDtypeStruct((n, hid), f32),
                   jax.ShapeDtypeStruct((1, hid), f32),
                   jax.ShapeDtypeStruct((1, hid), f32)],
    )(x_cont, x_cat, embed_charge, embed_pdgid, W_cont, bc2, W_cat, bk2,
      W_all, ba2)

    mean1 = s1s / n
    var1 = s1ss / n - mean1 * mean1
    sc1 = g_all.reshape(1, hid) / jnp.sqrt(var1 + 1e-5)
    sh1 = be_all.reshape(1, hid) - mean1 * sc1

    emb, a_rows, b_rows = pl.pallas_call(
        _s2_body,
        grid=(grid,),
        in_specs=[row(hid), full((1, hid)), full((1, hid)),
                  full((2 * hid, hid)), full((1, hid))],
        out_specs=[row(hid), row(hid), row(hid)],
        out_shape=[jax.ShapeDtypeStruct((n, hid), f32),
                   jax.ShapeDtypeStruct((n, hid), f32),
                   jax.ShapeDtypeStruct((n, hid), f32)],
    )(z, sc1, sh1, W_msg, bm2)

    m_rows = _segmax_sc(b_rows, src, dst, n, hid)

    s2s, s2ss = pl.pallas_call(
        _s4_body,
        grid=(grid,),
        in_specs=[row(hid), row(hid)],
        out_specs=[full((1, hid)), full((1, hid))],
        out_shape=[jax.ShapeDtypeStruct((1, hid), f32),
                   jax.ShapeDtypeStruct((1, hid), f32)],
    )(a_rows, m_rows)

    mean2 = s2s / n
    var2 = s2ss / n - mean2 * mean2
    sc2 = g_conv.reshape(1, hid) / jnp.sqrt(var2 + 1e-5)
    sh2 = be_conv.reshape(1, hid) - mean2 * sc2

    out = pl.pallas_call(
        _s5_body,
        grid=(grid,),
        in_specs=[row(hid), row(hid), row(hid), full((1, hid)),
                  full((1, hid)), full((hid, h2)), full((1, h2)),
                  full((1, h2)), full((1, 1))],
        out_specs=row(1),
        out_shape=jax.ShapeDtypeStruct((n, 1), f32),
    )(emb, a_rows, m_rows, sc2, sh2, W_out1, bo1, wo2, bo2)

    return out.reshape(n)


# speedup vs baseline: 6.5369x; 1.3408x over previous
import functools

import jax
import jax.numpy as jnp
from jax import lax
from jax.experimental import pallas as pl
from jax.experimental.pallas import tpu as pltpu
from jax.experimental.pallas import tpu_sc as plsc

_PDGS = (1, 2, 11, 13, 22, 130, 211)
_NEG = -3.0e38
_BLK = 4000
_NWORK = 32
_CHUNK = 1600
_SUB = 256


def _elu(x):
    return jnp.where(x > 0, x, jnp.exp(x) - 1.0)


def _dot(a, b):
    return jnp.dot(a, b, preferred_element_type=jnp.float32,
                   precision=lax.Precision.HIGHEST)


def _s1_body(xc_ref, cat_ref, t16_ref, wc_ref, bc_ref, wk_ref,
             bk_ref, wa_ref, ba_ref, z_ref, s_ref, ss_ref):
    h2 = wc_ref.shape[1]
    ec = _elu(_dot(xc_ref[...], wc_ref[...]) + bc_ref[...])
    cat = cat_ref[...]
    chrg = jnp.clip(cat[:, 1:2] + 1, 0, 2)
    p = jnp.abs(cat[:, 0:1])
    for i, v in enumerate(_PDGS):
        p = jnp.where(p == v, i, p)
    p = jnp.clip(p, 0, 6)
    iota2 = lax.broadcasted_iota(jnp.int32, (1, h2), 1)
    oh = ((chrg == iota2) | (p + 3 == iota2)).astype(jnp.float32)
    ecat = _elu(_dot(oh, _dot(t16_ref[...], wk_ref[...])) + bk_ref[...])
    wa = wa_ref[...]
    z = _elu(_dot(ecat, wa[0:h2, :]) + _dot(ec, wa[h2:2 * h2, :]) + ba_ref[...])
    z_ref[...] = z

    @pl.when(pl.program_id(0) == 0)
    def _():
        s_ref[...] = jnp.zeros_like(s_ref)
        ss_ref[...] = jnp.zeros_like(ss_ref)

    s_ref[...] += jnp.sum(z, axis=0, keepdims=True)
    ss_ref[...] += jnp.sum(z * z, axis=0, keepdims=True)


def _s2_body(z_ref, s1_ref, t1_ref, wm_ref, bm_ref, e_ref, a_ref, b_ref):
    hid = z_ref.shape[1]
    e = z_ref[...] * s1_ref[...] + t1_ref[...]
    wm = wm_ref[...]
    wt = wm[0:hid, :]
    wb = wm[hid:2 * hid, :]
    e_ref[...] = e
    b_ref[...] = _dot(e, wb)
    a_ref[...] = _dot(e, wt - wb) + bm_ref[...]


def _s4_body(a_ref, m_ref, s_ref, ss_ref):
    m = m_ref[...]
    agg = jnp.where(m[:, 0:1] > -1.0e37, a_ref[...] + m, 0.0)

    @pl.when(pl.program_id(0) == 0)
    def _():
        s_ref[...] = jnp.zeros_like(s_ref)
        ss_ref[...] = jnp.zeros_like(ss_ref)

    s_ref[...] += jnp.sum(agg, axis=0, keepdims=True)
    ss_ref[...] += jnp.sum(agg * agg, axis=0, keepdims=True)


def _s5_body(e_ref, a_ref, m_ref, s2_ref, t2_ref, wo1_ref, bo1_ref, wo2_ref,
             bo2_ref, o_ref):
    m = m_ref[...]
    agg = jnp.where(m[:, 0:1] > -1.0e37, a_ref[...] + m, 0.0)
    e2 = e_ref[...] + agg * s2_ref[...] + t2_ref[...]
    h = _elu(_dot(e2, wo1_ref[...]) + bo1_ref[...])
    o_ref[...] = jnp.sum(h * wo2_ref[...], axis=1, keepdims=True) + bo2_ref[...]


def _segmax_sc(b_rows, src, dst, n, hid):
    nrange = (-(-n // _NWORK) + 7) // 8 * 8
    nlast = n - (_NWORK - 1) * nrange
    e = src.shape[0]
    nch = e // _CHUNK
    sbits = max(nrange.bit_length(), 1)
    ldmask = (1 << sbits) - 1
    scap = (_SUB + _CHUNK + 16 + 63) // 64 * 64
    h2 = hid // 2
    mesh = plsc.VectorSubcoreMesh(core_axis_name="c", subcore_axis_name="s")
    cp = pltpu.CompilerParams(needs_layout_passes=False,
                              use_tc_tiling_on_sc=False)

    @functools.partial(
        pl.kernel,
        out_type=jax.ShapeDtypeStruct((n, hid), jnp.float32),
        mesh=mesh,
        compiler_params=cp,
        scratch_types=[
            pltpu.VMEM((nrange + 1, hid), jnp.float32),
            pltpu.VMEM((_CHUNK,), jnp.int32),
            pltpu.VMEM((_CHUNK,), jnp.int32),
            pltpu.VMEM((_CHUNK,), jnp.int32),
            pltpu.VMEM((_CHUNK,), jnp.int32),
            pltpu.VMEM((scap,), jnp.int32),
            pltpu.VMEM((scap,), jnp.int32),
            pltpu.VMEM((_SUB, hid), jnp.float32),
            pltpu.VMEM((_SUB, hid), jnp.float32),
            pltpu.SemaphoreType.DMA,
            pltpu.SemaphoreType.DMA,
            pltpu.SemaphoreType.DMA,
            pltpu.SemaphoreType.DMA,
            pltpu.SemaphoreType.DMA,
            pltpu.SemaphoreType.DMA,
        ],
    )
    def k(b_hbm, src_hbm, dst_hbm, m_hbm, acc, dbuf0, dbuf1, sbuf0, sbuf1,
          mpack, msrc, rows0, rows1, semd0, semd1, sems0, sems1, semg0,
          semg1):
        cid = lax.axis_index("c")
        sid = lax.axis_index("s")
        wid = sid * 2 + cid
        lo = pl.multiple_of(wid * nrange, 8)
        iota = lax.iota(jnp.int32, 16)
        negv = jnp.full((16,), _NEG, jnp.float32)

        @pl.loop(0, nrange + 1)
        def _(r):
            acc[r, pl.ds(0, 16)] = negv
            acc[r, pl.ds(16, 16)] = negv

        @pl.loop(0, scap // 16)
        def _(g):
            msrc[pl.ds(g * 16, 16)] = g * 16 + iota

        zero16 = jnp.zeros((16,), jnp.int32)

        def idx_start(c, db, sb, semd, sems):
            base = pl.multiple_of(lax.rem(c, nch) * _CHUNK, 8)
            pltpu.make_async_copy(dst_hbm.at[pl.ds(base, _CHUNK)], db,
                                  semd).start()
            pltpu.make_async_copy(src_hbm.at[pl.ds(base, _CHUNK)], sb,
                                  sems).start()

        def idx_wait(db, sb, semd, sems):
            pltpu.make_async_copy(dst_hbm.at[pl.ds(0, _CHUNK)], db,
                                  semd).wait()
            pltpu.make_async_copy(src_hbm.at[pl.ds(0, _CHUNK)], sb,
                                  sems).wait()

        def g_copy(t2, rb, sg):
            t2a = pl.multiple_of(t2, 8)
            return pltpu.make_async_copy(
                b_hbm.at[msrc.at[pl.ds(t2a, _SUB)]], rb, sg)

        def maxbatch(t2, rb, ngroups):
            def mx(j, u):
                rbase = j * 16
                base = pl.multiple_of(t2 + rbase, 8)
                ldvec = mpack[pl.ds(base, 16)] & ldmask
                idxs = [ldvec[t] for t in range(16)]
                for t in range(16):
                    ldx = idxs[t]
                    acc[ldx, pl.ds(0, h2)] = jnp.maximum(
                        acc[ldx, pl.ds(0, h2)],
                        rb[rbase + t, pl.ds(0, h2)])
                    acc[ldx, pl.ds(h2, h2)] = jnp.maximum(
                        acc[ldx, pl.ds(h2, h2)],
                        rb[rbase + t, pl.ds(h2, h2)])
                return u

            lax.fori_loop(0, ngroups, mx, 0)

        def process(db, sb, top):
            topv = jnp.zeros((16,), jnp.int32) + top

            def scan4(q, cv):
                qb = q * 64
                data = []
                for u in range(4):
                    d = db[pl.ds(qb + u * 16, 16)]
                    ld = d - lo
                    m = (ld >= 0) & (ld < nrange)
                    data.append((ld, m))
                pcs = [plsc.all_reduce_population_count(m) for _, m in data]
                css = [plsc.cumsum(m.astype(jnp.int32)) for _, m in data]
                for u in range(4):
                    ld, m = data[u]
                    pos = cv + css[u] - 1
                    epos = (qb + u * 16) + iota
                    plsc.store_scatter(mpack, [pos], (epos << sbits) | ld,
                                       mask=m)
                    cv = cv + pcs[u]
                return cv

            cv = lax.fori_loop(0, _CHUNK // 64, scan4, topv)
            tc_ = jnp.max(cv)
            pad = (16 - (tc_ & 15)) & 15
            sent = (wid << sbits) | nrange
            plsc.store_scatter(mpack, [tc_ + iota], zero16 + sent,
                               mask=iota < pad)
            topp = tc_ + pad

            def unp(g, u):
                base = pl.multiple_of(top + g * 16, 8)
                pv = mpack[pl.ds(base, 16)]
                ep = lax.shift_right_logical(pv, sbits)
                msrc[pl.ds(base, 16)] = plsc.load_gather(sb, [ep])
                return u

            lax.fori_loop(0, (topp - top) // 16, unp, 0)

            def d2(t):
                ca = g_copy(t - _SUB, rows0, semg0)
                cb = g_copy(t - 2 * _SUB, rows1, semg1)
                ca.start()
                cb.start()
                ca.wait()
                maxbatch(t - _SUB, rows0, _SUB // 16)
                cb.wait()
                maxbatch(t - 2 * _SUB, rows1, _SUB // 16)
                return t - 2 * _SUB

            t2_ = lax.while_loop(lambda t: t >= 2 * _SUB, d2, topp)

            def d1(t):
                c1 = g_copy(t - _SUB, rows0, semg0)
                c1.start()
                c1.wait()
                maxbatch(t - _SUB, rows0, _SUB // 16)
                return t - _SUB

            return lax.while_loop(lambda t: t >= _SUB, d1, t2_)

        idx_start(0, dbuf0, sbuf0, semd0, sems0)

        def chunk_pair(i, top):
            c = i * 2
            idx_start(c + 1, dbuf1, sbuf1, semd1, sems1)
            idx_wait(dbuf0, sbuf0, semd0, sems0)
            top = process(dbuf0, sbuf0, top)
            idx_start(c + 2, dbuf0, sbuf0, semd0, sems0)
            idx_wait(dbuf1, sbuf1, semd1, sems1)
            top = process(dbuf1, sbuf1, top)
            return top

        top = lax.fori_loop(0, nch // 2, chunk_pair, jnp.int32(0))
        idx_wait(dbuf0, sbuf0, semd0, sems0)

        @pl.when(top > 0)
        def _():
            c1 = pltpu.make_async_copy(
                b_hbm.at[msrc.at[pl.ds(0, _SUB)]], rows0, semg0)
            c1.start()
            c1.wait()
            maxbatch(0, rows0, top // 16)

        @pl.when(wid < _NWORK - 1)
        def _():
            pltpu.sync_copy(acc.at[pl.ds(0, nrange)],
                            m_hbm.at[pl.ds(lo, nrange)])

        @pl.when(wid == _NWORK - 1)
        def _():
            pltpu.sync_copy(acc.at[pl.ds(0, nlast)],
                            m_hbm.at[pl.ds(lo, nlast)])

    return k(b_rows, src, dst)


def kernel(x_cont, x_cat, edge_index, batch, embed_charge, embed_pdgid,
           W_cont, b_cont, W_cat, b_cat, W_all, b_all, g_all, be_all,
           W_msg, b_msg, g_conv, be_conv, W_out1, b_out1, W_out2, b_out2):
    n, cont = x_cont.shape
    hid = W_all.shape[0]
    h4 = hid // 4
    h2 = hid // 2
    grid = n // _BLK
    f32 = jnp.float32

    x_cat = x_cat.astype(jnp.int32)
    src = edge_index[0].astype(jnp.int32)
    dst = edge_index[1].astype(jnp.int32)

    bc2 = b_cont.reshape(1, h2)
    bk2 = b_cat.reshape(1, h2)
    ba2 = b_all.reshape(1, hid)
    bm2 = b_msg.reshape(1, hid)
    bo1 = b_out1.reshape(1, h2)
    wo2 = W_out2.reshape(1, h2)
    bo2 = b_out2.reshape(1, 1)

    full = lambda s: pl.BlockSpec(s, lambda i: (0, 0))
    row = lambda c: pl.BlockSpec((_BLK, c), lambda i: (i, 0))

    t16 = jnp.zeros((h2, h2), jnp.float32)
    t16 = t16.at[0:3, 0:h4].set(embed_charge)
    t16 = t16.at[3:10, h4:2 * h4].set(embed_pdgid)

    z, s1s, s1ss = pl.pallas_call(
        _s1_body,
        grid=(grid,),
        in_specs=[row(cont), row(2), full((h2, h2)),
                  full((cont, h2)), full((1, h2)), full((h2, h2)),
                  full((1, h2)), full((hid, hid)), full((1, hid))],
        out_specs=[row(hid), full((1, hid)), full((1, hid))],
        out_shape=[jax.ShapeDtypeStruct((n, hid), f32),
                   jax.ShapeDtypeStruct((1, hid), f32),
                   jax.ShapeDtypeStruct((1, hid), f32)],
    )(x_cont, x_cat, t16, W_cont, bc2, W_cat, bk2, W_all, ba2)

    mean1 = s1s / n
    var1 = s1ss / n - mean1 * mean1
    sc1 = g_all.reshape(1, hid) / jnp.sqrt(var1 + 1e-5)
    sh1 = be_all.reshape(1, hid) - mean1 * sc1

    emb, a_rows, b_rows = pl.pallas_call(
        _s2_body,
        grid=(grid,),
        in_specs=[row(hid), full((1, hid)), full((1, hid)),
                  full((2 * hid, hid)), full((1, hid))],
        out_specs=[row(hid), row(hid), row(hid)],
        out_shape=[jax.ShapeDtypeStruct((n, hid), f32),
                   jax.ShapeDtypeStruct((n, hid), f32),
                   jax.ShapeDtypeStruct((n, hid), f32)],
    )(z, sc1, sh1, W_msg, bm2)

    m_rows = _segmax_sc(b_rows, src, dst, n, hid)

    s2s, s2ss = pl.pallas_call(
        _s4_body,
        grid=(grid,),
        in_specs=[row(hid), row(hid)],
        out_specs=[full((1, hid)), full((1, hid))],
        out_shape=[jax.ShapeDtypeStruct((1, hid), f32),
                   jax.ShapeDtypeStruct((1, hid), f32)],
    )(a_rows, m_rows)

    mean2 = s2s / n
    var2 = s2ss / n - mean2 * mean2
    sc2 = g_conv.reshape(1, hid) / jnp.sqrt(var2 + 1e-5)
    sh2 = be_conv.reshape(1, hid) - mean2 * sc2

    out = pl.pallas_call(
        _s5_body,
        grid=(grid,),
        in_specs=[row(hid), row(hid), row(hid), full((1, hid)),
                  full((1, hid)), full((hid, h2)), full((1, h2)),
                  full((1, h2)), full((1, 1))],
        out_specs=row(1),
        out_shape=jax.ShapeDtypeStruct((n, 1), f32),
    )(emb, a_rows, m_rows, sc2, sh2, W_out1, bo1, wo2, bo2)

    return out.reshape(n)


# speedup vs baseline: 7.3469x; 1.1239x over previous
import functools

import jax
import jax.numpy as jnp
from jax import lax
from jax.experimental import pallas as pl
from jax.experimental.pallas import tpu as pltpu
from jax.experimental.pallas import tpu_sc as plsc

_PDGS = (1, 2, 11, 13, 22, 130, 211)
_NEG = -3.0e38
_BLK = 4000
_NWORK = 32
_CHUNK = 1600
_SUB = 256


def _elu(x):
    return jnp.where(x > 0, x, jnp.exp(x) - 1.0)


def _dot(a, b):
    return jnp.dot(a, b, preferred_element_type=jnp.float32)


def _s1_body(xc_ref, cat_ref, t16_ref, wc_ref, bc_ref, wk_ref,
             bk_ref, wa_ref, ba_ref, z_ref, s_ref, ss_ref):
    h2 = wc_ref.shape[1]
    ec = _elu(_dot(xc_ref[...], wc_ref[...]) + bc_ref[...])
    cat = cat_ref[...]
    chrg = jnp.clip(cat[:, 1:2] + 1, 0, 2)
    p = jnp.abs(cat[:, 0:1])
    for i, v in enumerate(_PDGS):
        p = jnp.where(p == v, i, p)
    p = jnp.clip(p, 0, 6)
    iota2 = lax.broadcasted_iota(jnp.int32, (1, h2), 1)
    oh = ((chrg == iota2) | (p + 3 == iota2)).astype(jnp.float32)
    ecat = _elu(_dot(oh, _dot(t16_ref[...], wk_ref[...])) + bk_ref[...])
    wa = wa_ref[...]
    z = _elu(_dot(ecat, wa[0:h2, :]) + _dot(ec, wa[h2:2 * h2, :]) + ba_ref[...])
    z_ref[...] = z

    @pl.when(pl.program_id(0) == 0)
    def _():
        s_ref[...] = jnp.zeros_like(s_ref)
        ss_ref[...] = jnp.zeros_like(ss_ref)

    s_ref[...] += jnp.sum(z, axis=0, keepdims=True)
    ss_ref[...] += jnp.sum(z * z, axis=0, keepdims=True)


def _s2_body(z_ref, s1_ref, t1_ref, wm_ref, bm_ref, e_ref, a_ref, b_ref):
    hid = z_ref.shape[1]
    e = z_ref[...] * s1_ref[...] + t1_ref[...]
    wm = wm_ref[...]
    wt = wm[0:hid, :]
    wb = wm[hid:2 * hid, :]
    e_ref[...] = e
    b_ref[...] = _dot(e, wb)
    a_ref[...] = _dot(e, wt - wb) + bm_ref[...]


def _s4_body(a_ref, m_ref, s_ref, ss_ref):
    m = m_ref[...]
    agg = jnp.where(m[:, 0:1] > -1.0e37, a_ref[...] + m, 0.0)

    @pl.when(pl.program_id(0) == 0)
    def _():
        s_ref[...] = jnp.zeros_like(s_ref)
        ss_ref[...] = jnp.zeros_like(ss_ref)

    s_ref[...] += jnp.sum(agg, axis=0, keepdims=True)
    ss_ref[...] += jnp.sum(agg * agg, axis=0, keepdims=True)


def _s5_body(e_ref, a_ref, m_ref, s2_ref, t2_ref, wo1_ref, bo1_ref, wo2_ref,
             bo2_ref, o_ref):
    m = m_ref[...]
    agg = jnp.where(m[:, 0:1] > -1.0e37, a_ref[...] + m, 0.0)
    e2 = e_ref[...] + agg * s2_ref[...] + t2_ref[...]
    h = _elu(_dot(e2, wo1_ref[...]) + bo1_ref[...])
    o_ref[...] = jnp.sum(h * wo2_ref[...], axis=1, keepdims=True) + bo2_ref[...]


def _segmax_sc(b_rows, src, dst, n, hid):
    nrange = (-(-n // _NWORK) + 7) // 8 * 8
    nlast = n - (_NWORK - 1) * nrange
    e = src.shape[0]
    nch = e // _CHUNK
    sbits = max(nrange.bit_length(), 1)
    ldmask = (1 << sbits) - 1
    scap = (_SUB + _CHUNK + 16 + 63) // 64 * 64
    h2 = hid // 2
    mesh = plsc.VectorSubcoreMesh(core_axis_name="c", subcore_axis_name="s")
    cp = pltpu.CompilerParams(needs_layout_passes=False,
                              use_tc_tiling_on_sc=False)

    @functools.partial(
        pl.kernel,
        out_type=jax.ShapeDtypeStruct((n, hid), jnp.float32),
        mesh=mesh,
        compiler_params=cp,
        scratch_types=[
            pltpu.VMEM((nrange + 1, hid), jnp.float32),
            pltpu.VMEM((_CHUNK,), jnp.int32),
            pltpu.VMEM((_CHUNK,), jnp.int32),
            pltpu.VMEM((_CHUNK,), jnp.int32),
            pltpu.VMEM((_CHUNK,), jnp.int32),
            pltpu.VMEM((scap,), jnp.int32),
            pltpu.VMEM((scap,), jnp.int32),
            pltpu.VMEM((_SUB, hid), jnp.float32),
            pltpu.VMEM((_SUB, hid), jnp.float32),
            pltpu.SemaphoreType.DMA,
            pltpu.SemaphoreType.DMA,
            pltpu.SemaphoreType.DMA,
            pltpu.SemaphoreType.DMA,
            pltpu.SemaphoreType.DMA,
            pltpu.SemaphoreType.DMA,
        ],
    )
    def k(b_hbm, src_hbm, dst_hbm, m_hbm, acc, dbuf0, dbuf1, sbuf0, sbuf1,
          mpack, msrc, rows0, rows1, semd0, semd1, sems0, sems1, semg0,
          semg1):
        cid = lax.axis_index("c")
        sid = lax.axis_index("s")
        wid = sid * 2 + cid
        lo = pl.multiple_of(wid * nrange, 8)
        iota = lax.iota(jnp.int32, 16)
        negv = jnp.full((16,), _NEG, jnp.float32)

        @pl.loop(0, nrange + 1)
        def _(r):
            acc[r, pl.ds(0, 16)] = negv
            acc[r, pl.ds(16, 16)] = negv

        @pl.loop(0, scap // 16)
        def _(g):
            msrc[pl.ds(g * 16, 16)] = g * 16 + iota

        zero16 = jnp.zeros((16,), jnp.int32)

        def idx_start(c, db, sb, semd, sems):
            base = pl.multiple_of(lax.rem(c, nch) * _CHUNK, 8)
            pltpu.make_async_copy(dst_hbm.at[pl.ds(base, _CHUNK)], db,
                                  semd).start()
            pltpu.make_async_copy(src_hbm.at[pl.ds(base, _CHUNK)], sb,
                                  sems).start()

        def idx_wait(db, sb, semd, sems):
            pltpu.make_async_copy(dst_hbm.at[pl.ds(0, _CHUNK)], db,
                                  semd).wait()
            pltpu.make_async_copy(src_hbm.at[pl.ds(0, _CHUNK)], sb,
                                  sems).wait()

        def g_copy(t2, rb, sg):
            t2a = pl.multiple_of(t2, 8)
            return pltpu.make_async_copy(
                b_hbm.at[msrc.at[pl.ds(t2a, _SUB)]], rb, sg)

        def maxbatch(t2, rb, ngroups):
            def mx(j, u):
                rbase = j * 16
                base = pl.multiple_of(t2 + rbase, 8)
                ldvec = mpack[pl.ds(base, 16)] & ldmask
                idxs = [ldvec[t] for t in range(16)]
                for t in range(16):
                    ldx = idxs[t]
                    acc[ldx, pl.ds(0, h2)] = jnp.maximum(
                        acc[ldx, pl.ds(0, h2)],
                        rb[rbase + t, pl.ds(0, h2)])
                    acc[ldx, pl.ds(h2, h2)] = jnp.maximum(
                        acc[ldx, pl.ds(h2, h2)],
                        rb[rbase + t, pl.ds(h2, h2)])
                return u

            lax.fori_loop(0, ngroups, mx, 0)

        def process(db, sb, top):
            topv = jnp.zeros((16,), jnp.int32) + top

            def scan4(q, cv):
                qb = q * 64
                data = []
                for u in range(4):
                    d = db[pl.ds(qb + u * 16, 16)]
                    ld = d - lo
                    m = (ld >= 0) & (ld < nrange)
                    data.append((ld, m))
                pcs = [plsc.all_reduce_population_count(m) for _, m in data]
                css = [plsc.cumsum(m.astype(jnp.int32)) for _, m in data]
                for u in range(4):
                    ld, m = data[u]
                    pos = cv + css[u] - 1
                    epos = (qb + u * 16) + iota
                    plsc.store_scatter(mpack, [pos], (epos << sbits) | ld,
                                       mask=m)
                    cv = cv + pcs[u]
                return cv

            cv = lax.fori_loop(0, _CHUNK // 64, scan4, topv)
            tc_ = jnp.max(cv)
            pad = (16 - (tc_ & 15)) & 15
            sent = (wid << sbits) | nrange
            plsc.store_scatter(mpack, [tc_ + iota], zero16 + sent,
                               mask=iota < pad)
            topp = tc_ + pad

            def unp(g, u):
                base = pl.multiple_of(top + g * 16, 8)
                pv = mpack[pl.ds(base, 16)]
                ep = lax.shift_right_logical(pv, sbits)
                msrc[pl.ds(base, 16)] = plsc.load_gather(sb, [ep])
                return u

            lax.fori_loop(0, (topp - top) // 16, unp, 0)

            def d2(t):
                ca = g_copy(t - _SUB, rows0, semg0)
                cb = g_copy(t - 2 * _SUB, rows1, semg1)
                ca.start()
                cb.start()
                ca.wait()
                maxbatch(t - _SUB, rows0, _SUB // 16)
                cb.wait()
                maxbatch(t - 2 * _SUB, rows1, _SUB // 16)
                return t - 2 * _SUB

            t2_ = lax.while_loop(lambda t: t >= 2 * _SUB, d2, topp)

            def d1(t):
                c1 = g_copy(t - _SUB, rows0, semg0)
                c1.start()
                c1.wait()
                maxbatch(t - _SUB, rows0, _SUB // 16)
                return t - _SUB

            return lax.while_loop(lambda t: t >= _SUB, d1, t2_)

        idx_start(0, dbuf0, sbuf0, semd0, sems0)

        def chunk_pair(i, top):
            c = i * 2
            idx_start(c + 1, dbuf1, sbuf1, semd1, sems1)
            idx_wait(dbuf0, sbuf0, semd0, sems0)
            top = process(dbuf0, sbuf0, top)
            idx_start(c + 2, dbuf0, sbuf0, semd0, sems0)
            idx_wait(dbuf1, sbuf1, semd1, sems1)
            top = process(dbuf1, sbuf1, top)
            return top

        top = lax.fori_loop(0, nch // 2, chunk_pair, jnp.int32(0))
        idx_wait(dbuf0, sbuf0, semd0, sems0)

        @pl.when(top > 0)
        def _():
            c1 = pltpu.make_async_copy(
                b_hbm.at[msrc.at[pl.ds(0, _SUB)]], rows0, semg0)
            c1.start()
            c1.wait()
            maxbatch(0, rows0, top // 16)

        @pl.when(wid < _NWORK - 1)
        def _():
            pltpu.sync_copy(acc.at[pl.ds(0, nrange)],
                            m_hbm.at[pl.ds(lo, nrange)])

        @pl.when(wid == _NWORK - 1)
        def _():
            pltpu.sync_copy(acc.at[pl.ds(0, nlast)],
                            m_hbm.at[pl.ds(lo, nlast)])

    return k(b_rows, src, dst)


def kernel(x_cont, x_cat, edge_index, batch, embed_charge, embed_pdgid,
           W_cont, b_cont, W_cat, b_cat, W_all, b_all, g_all, be_all,
           W_msg, b_msg, g_conv, be_conv, W_out1, b_out1, W_out2, b_out2):
    n, cont = x_cont.shape
    hid = W_all.shape[0]
    h4 = hid // 4
    h2 = hid // 2
    grid = n // _BLK
    f32 = jnp.float32

    x_cat = x_cat.astype(jnp.int32)
    src = edge_index[0].astype(jnp.int32)
    dst = edge_index[1].astype(jnp.int32)

    bc2 = b_cont.reshape(1, h2)
    bk2 = b_cat.reshape(1, h2)
    ba2 = b_all.reshape(1, hid)
    bm2 = b_msg.reshape(1, hid)
    bo1 = b_out1.reshape(1, h2)
    wo2 = W_out2.reshape(1, h2)
    bo2 = b_out2.reshape(1, 1)

    full = lambda s: pl.BlockSpec(s, lambda i: (0, 0))
    row = lambda c: pl.BlockSpec((_BLK, c), lambda i: (i, 0))

    t16 = jnp.zeros((h2, h2), jnp.float32)
    t16 = t16.at[0:3, 0:h4].set(embed_charge)
    t16 = t16.at[3:10, h4:2 * h4].set(embed_pdgid)

    z, s1s, s1ss = pl.pallas_call(
        _s1_body,
        grid=(grid,),
        in_specs=[row(cont), row(2), full((h2, h2)),
                  full((cont, h2)), full((1, h2)), full((h2, h2)),
                  full((1, h2)), full((hid, hid)), full((1, hid))],
        out_specs=[row(hid), full((1, hid)), full((1, hid))],
        out_shape=[jax.ShapeDtypeStruct((n, hid), f32),
                   jax.ShapeDtypeStruct((1, hid), f32),
                   jax.ShapeDtypeStruct((1, hid), f32)],
    )(x_cont, x_cat, t16, W_cont, bc2, W_cat, bk2, W_all, ba2)

    mean1 = s1s / n
    var1 = s1ss / n - mean1 * mean1
    sc1 = g_all.reshape(1, hid) / jnp.sqrt(var1 + 1e-5)
    sh1 = be_all.reshape(1, hid) - mean1 * sc1

    emb, a_rows, b_rows = pl.pallas_call(
        _s2_body,
        grid=(grid,),
        in_specs=[row(hid), full((1, hid)), full((1, hid)),
                  full((2 * hid, hid)), full((1, hid))],
        out_specs=[row(hid), row(hid), row(hid)],
        out_shape=[jax.ShapeDtypeStruct((n, hid), f32),
                   jax.ShapeDtypeStruct((n, hid), f32),
                   jax.ShapeDtypeStruct((n, hid), f32)],
    )(z, sc1, sh1, W_msg, bm2)

    m_rows = _segmax_sc(b_rows, src, dst, n, hid)

    s2s, s2ss = pl.pallas_call(
        _s4_body,
        grid=(grid,),
        in_specs=[row(hid), row(hid)],
        out_specs=[full((1, hid)), full((1, hid))],
        out_shape=[jax.ShapeDtypeStruct((1, hid), f32),
                   jax.ShapeDtypeStruct((1, hid), f32)],
    )(a_rows, m_rows)

    mean2 = s2s / n
    var2 = s2ss / n - mean2 * mean2
    sc2 = g_conv.reshape(1, hid) / jnp.sqrt(var2 + 1e-5)
    sh2 = be_conv.reshape(1, hid) - mean2 * sc2

    out = pl.pallas_call(
        _s5_body,
        grid=(grid,),
        in_specs=[row(hid), row(hid), row(hid), full((1, hid)),
                  full((1, hid)), full((hid, h2)), full((1, h2)),
                  full((1, h2)), full((1, 1))],
        out_specs=row(1),
        out_shape=jax.ShapeDtypeStruct((n, 1), f32),
    )(emb, a_rows, m_rows, sc2, sh2, W_out1, bo1, wo2, bo2)

    return out.reshape(n)
